# baseline (speedup 1.0000x reference)
import jax
import jax.numpy as jnp
from jax.experimental import pallas as pl

_EMB = 64
_DIN = 32
_K = _DIN + 8


def _fused_kernel(seg_ref, x_ref, lhs_ref, o_ref):
    x = x_ref[...]
    lhs = lhs_ref[...]
    seg = seg_ref[...]

    pt = x.shape[0]
    bsz = x.shape[2]
    onehot = (seg == jax.lax.broadcasted_iota(
        jnp.int32, (pt, _K - _DIN, bsz), 1)).astype(jnp.float32)
    rhs = jnp.concatenate([x, onehot], axis=1)
    lhsb = jnp.broadcast_to(lhs[None], (pt, _EMB, _K))
    o_ref[...] = jax.lax.dot_general(
        lhsb, rhs, (((2,), (1,)), ((0,), (0,))),
        preferred_element_type=jnp.float32)


@jax.jit
def kernel(x, W, b, table, seg):
    B, P, DIN = x.shape
    PT = 62
    BH = B // 2

    xt = jnp.transpose(x, (1, 2, 0))
    tb = table + b[None, :]
    lhs = jnp.concatenate(
        [W.T, tb.T, jnp.zeros((_EMB, _K - _DIN - table.shape[0]),
                              jnp.float32)], axis=1)
    seg3 = seg.reshape(P, 1, 1)

    grid = (P // PT, 2)
    out_t = pl.pallas_call(
        _fused_kernel,
        grid=grid,
        in_specs=[
            pl.BlockSpec((PT, 1, 1), lambda i, j: (i, 0, 0)),
            pl.BlockSpec((PT, DIN, BH), lambda i, j: (i, 0, j)),
            pl.BlockSpec((_EMB, _K), lambda i, j: (0, 0)),
        ],
        out_specs=pl.BlockSpec((PT, _EMB, BH), lambda i, j: (i, 0, j)),
        out_shape=jax.ShapeDtypeStruct((P, _EMB, B), jnp.float32),
    )(seg3, xt, lhs)
    return jnp.transpose(out_t, (2, 0, 1))


# speedup vs baseline: 1.0259x; 1.0259x over previous
import jax
import jax.numpy as jnp
from jax.experimental import pallas as pl

_EMB = 64
_DIN = 32
_K = _DIN + 8


def _fused_kernel(seg_ref, x_ref, lhs_ref, o_ref):
    x = x_ref[...]
    lhs = lhs_ref[...]
    seg = seg_ref[...]

    pt = x.shape[0]
    bsz = x.shape[2]
    onehot = (seg == jax.lax.broadcasted_iota(
        jnp.int32, (pt, _K - _DIN, bsz), 1)).astype(jnp.float32)
    rhs = jnp.concatenate([x, onehot], axis=1)
    lhsb = jnp.broadcast_to(lhs[None], (pt, _EMB, _K))
    o_ref[...] = jax.lax.dot_general(
        lhsb, rhs, (((2,), (1,)), ((0,), (0,))),
        preferred_element_type=jnp.float32)


@jax.jit
def kernel(x, W, b, table, seg):
    B, P, DIN = x.shape
    PT = 62

    xt = jnp.transpose(x, (1, 2, 0))
    tb = table + b[None, :]
    lhs = jnp.concatenate(
        [W.T, tb.T, jnp.zeros((_EMB, _K - _DIN - table.shape[0]),
                              jnp.float32)], axis=1)
    seg3 = seg.reshape(P, 1, 1)

    grid = (P // PT,)
    out_t = pl.pallas_call(
        _fused_kernel,
        grid=grid,
        in_specs=[
            pl.BlockSpec((PT, 1, 1), lambda i: (i, 0, 0)),
            pl.BlockSpec((PT, DIN, B), lambda i: (i, 0, 0)),
            pl.BlockSpec((_EMB, _K), lambda i: (0, 0)),
        ],
        out_specs=pl.BlockSpec((PT, _EMB, B), lambda i: (i, 0, 0)),
        out_shape=jax.ShapeDtypeStruct((P, _EMB, B), jnp.float32),
    )(seg3, xt, lhs)
    return jnp.transpose(out_t, (2, 0, 1))
